# Initial kernel scaffold; baseline (speedup 1.0000x reference)
import jax
import jax.numpy as jnp
from jax.experimental import pallas as pl


def kernel(feats_emb, center_coords, rand_scores, W1, b1, W2, b2, ln_gamma, ln_beta):
    raise NotImplementedError("write your pallas kernel here")


# speedup vs baseline: 4.0881x; 4.0881x over previous
import functools
import math

import jax
import jax.numpy as jnp
from jax import lax
from jax.experimental import pallas as pl
from jax.experimental.pallas import tpu as pltpu
from jax.experimental.pallas import tpu_sc as plsc

B, G, E = 64, 2048, 384
K = G // 2
V = G - K
NW = 32
ROWS_PER_W = B // NW
GCHUNK = 128
NCHUNK = V // GCHUNK


def _cumsum_lanes(x):
    sh = 1
    while sh < G:
        z = jnp.zeros((x.shape[0], sh), x.dtype)
        x = x + jnp.concatenate([z, x[:, :-sh]], axis=1)
        sh *= 2
    return x


def _posmap_body(scores_ref, posmap_ref):
    s = scores_ref[...]
    bits = lax.bitcast_convert_type(s, jnp.int32)

    def step(_, carry):
        lo, hi = carry
        mid = lo + ((hi - lo) >> 1)
        cnt = jnp.sum((bits > mid).astype(jnp.int32), axis=1, keepdims=True)
        pred = cnt < K
        return jnp.where(pred, lo, mid + 1), jnp.where(pred, mid, hi)

    lo0 = jnp.zeros((B, 1), jnp.int32)
    hi0 = jnp.full((B, 1), 0x7F800000, jnp.int32)
    t, _ = lax.fori_loop(0, 31, step, (lo0, hi0))

    gt = bits > t
    cnt_gt = jnp.sum(gt.astype(jnp.int32), axis=1, keepdims=True)
    need = K - cnt_gt
    tie = (bits == t).astype(jnp.int32)
    tie_excl = _cumsum_lanes(tie) - tie
    masked = gt | ((tie == 1) & (tie_excl < need))
    vis = 1 - masked.astype(jnp.int32)
    c = _cumsum_lanes(vis)
    posmap_ref[...] = jnp.where(masked, -1, c - 1)


def _posmap(rand_scores):
    return pl.pallas_call(
        _posmap_body,
        out_shape=jax.ShapeDtypeStruct((B, G), jnp.int32),
    )(rand_scores)


def _sc_body(posmap_hbm, coords_hbm, feats_hbm, out_feats_hbm, out_coords_hbm,
             pos_v, coords_v, idx_v, cvis_v, rows_v, sem):
    wid = lax.axis_index("s") * 2 + lax.axis_index("c")

    for r in range(ROWS_PER_W):
        b = wid * ROWS_PER_W + r
        pltpu.sync_copy(posmap_hbm.at[b], pos_v)
        pltpu.sync_copy(coords_hbm.at[b], coords_v)

        base_flat = b * G

        def chunk(ci, _):
            pos = pos_v[pl.ds(ci * 16, 16)]
            m = pos >= 0
            g = ci * 16 + lax.iota(jnp.int32, 16)
            plsc.store_scatter(idx_v, [pos], base_flat + g, mask=m)
            cvals = coords_v[pl.ds(ci * 16, 16)]
            plsc.store_scatter(cvis_v, [pos], cvals, mask=m)
            return 0

        lax.fori_loop(0, G // 16, chunk, 0)

        pltpu.sync_copy(cvis_v, out_coords_hbm.at[pl.ds(b * V, V)])

        for ch in range(NCHUNK):
            idx_slice = idx_v.at[pl.ds(ch * GCHUNK, GCHUNK)]
            pltpu.async_copy(feats_hbm.at[idx_slice], rows_v, sem).wait()
            pltpu.sync_copy(
                rows_v, out_feats_hbm.at[pl.ds(b * V + ch * GCHUNK, GCHUNK)])


@functools.partial(jax.jit, static_argnums=())
def _sc_gather(posmap, coords2d, feats_flat):
    mesh = plsc.VectorSubcoreMesh(core_axis_name="c", subcore_axis_name="s",
                                  num_cores=2, num_subcores=16)
    kern = pl.kernel(
        _sc_body,
        out_type=[
            jax.ShapeDtypeStruct((B * V, E), jnp.float32),
            jax.ShapeDtypeStruct((B * V,), jnp.float32),
        ],
        mesh=mesh,
        compiler_params=pltpu.CompilerParams(needs_layout_passes=False),
        scratch_types=[
            pltpu.VMEM((G,), jnp.int32),
            pltpu.VMEM((G,), jnp.float32),
            pltpu.VMEM((V,), jnp.int32),
            pltpu.VMEM((V,), jnp.float32),
            pltpu.VMEM((GCHUNK, E), jnp.float32),
            pltpu.SemaphoreType.DMA,
        ],
    )
    return kern(posmap, coords2d, feats_flat)


TBLK = 2048


def _mlp_body(cv_ref, w1_ref, b1_ref, w2_ref, b2_ref, g_ref, beta_ref, out_ref):
    c = cv_ref[...]
    h = c * w1_ref[...] + b1_ref[...]
    h = 0.5 * h * (1.0 + lax.erf(h * (1.0 / math.sqrt(2.0))))
    h2 = jnp.dot(h, w2_ref[...], preferred_element_type=jnp.float32)
    h2 = h2 + b2_ref[...]
    mean = jnp.mean(h2, axis=1, keepdims=True)
    ctr = h2 - mean
    var = jnp.mean(ctr * ctr, axis=1, keepdims=True)
    out_ref[...] = ctr / jnp.sqrt(var + 1e-5) * g_ref[...] + beta_ref[...]


def _mlp(cv, W1, b1, W2, b2, ln_gamma, ln_beta):
    nt = B * V
    grid = nt // TBLK
    return pl.pallas_call(
        _mlp_body,
        grid=(grid,),
        in_specs=[
            pl.BlockSpec((TBLK, 1), lambda i: (i, 0)),
            pl.BlockSpec((1, 128), lambda i: (0, 0)),
            pl.BlockSpec((1, 128), lambda i: (0, 0)),
            pl.BlockSpec((128, E), lambda i: (0, 0)),
            pl.BlockSpec((1, E), lambda i: (0, 0)),
            pl.BlockSpec((1, E), lambda i: (0, 0)),
            pl.BlockSpec((1, E), lambda i: (0, 0)),
        ],
        out_specs=pl.BlockSpec((TBLK, E), lambda i: (i, 0)),
        out_shape=jax.ShapeDtypeStruct((nt, E), jnp.float32),
    )(cv, W1, b1, W2, b2, ln_gamma, ln_beta)


def kernel(feats_emb, center_coords, rand_scores, W1, b1, W2, b2,
           ln_gamma, ln_beta):
    posmap = _posmap(rand_scores)
    coords2d = center_coords.reshape(B, G)
    feats_flat = feats_emb.reshape(B * G, E)
    fv_flat, cvis = _sc_gather(posmap, coords2d, feats_flat)
    pos_emb = _mlp(cvis.reshape(B * V, 1), W1, b1.reshape(1, 128), W2,
                   b2.reshape(1, E), ln_gamma.reshape(1, E),
                   ln_beta.reshape(1, E))
    return fv_flat.reshape(B, V, E), pos_emb.reshape(B, V, E)


# speedup vs baseline: 4.9389x; 1.2081x over previous
import functools
import math

import jax
import jax.numpy as jnp
from jax import lax
from jax.experimental import pallas as pl
from jax.experimental.pallas import tpu as pltpu
from jax.experimental.pallas import tpu_sc as plsc

B, G, E = 64, 2048, 384
K = G // 2
V = G - K
NW = 32
ROWS_PER_W = B // NW
GCHUNK = 128
NCHUNK = V // GCHUNK


def _cumsum_lanes(x):
    sh = 1
    while sh < G:
        z = jnp.zeros((x.shape[0], sh), x.dtype)
        x = x + jnp.concatenate([z, x[:, :-sh]], axis=1)
        sh *= 2
    return x


def _posmap_body(scores_ref, posmap_ref):
    s = scores_ref[...]
    bits = lax.bitcast_convert_type(s, jnp.int32)

    def step(_, carry):
        lo, hi = carry
        mid = lo + ((hi - lo) >> 1)
        cnt = jnp.sum((bits > mid).astype(jnp.int32), axis=1, keepdims=True)
        pred = cnt < K
        return jnp.where(pred, lo, mid + 1), jnp.where(pred, mid, hi)

    lo0 = jnp.zeros((B, 1), jnp.int32)
    hi0 = jnp.full((B, 1), 0x7F800000, jnp.int32)
    t, _ = lax.fori_loop(0, 31, step, (lo0, hi0))

    gt = bits > t
    cnt_gt = jnp.sum(gt.astype(jnp.int32), axis=1, keepdims=True)
    need = K - cnt_gt
    tie = (bits == t).astype(jnp.int32)
    tie_excl = _cumsum_lanes(tie) - tie
    masked = gt | ((tie == 1) & (tie_excl < need))
    vis = 1 - masked.astype(jnp.int32)
    c = _cumsum_lanes(vis)
    posmap_ref[...] = jnp.where(masked, -1, c - 1)


def _posmap(rand_scores):
    return pl.pallas_call(
        _posmap_body,
        out_shape=jax.ShapeDtypeStruct((B, G), jnp.int32),
    )(rand_scores)


def _sc_mesh():
    return plsc.VectorSubcoreMesh(core_axis_name="c", subcore_axis_name="s",
                                  num_cores=2, num_subcores=16)


def _sc_compact_body(posmap_hbm, coords_hbm, out_idx_hbm, out_coords_hbm,
                     pos_v, coords_v, idx_v, cvis_v):
    wid = lax.axis_index("s") * 2 + lax.axis_index("c")

    for r in range(ROWS_PER_W):
        b = wid * ROWS_PER_W + r
        pltpu.sync_copy(posmap_hbm.at[b], pos_v)
        pltpu.sync_copy(coords_hbm.at[b], coords_v)

        base_flat = b * G

        def chunk(ci, _):
            pos = pos_v[pl.ds(ci * 16, 16)]
            m = pos >= 0
            g = ci * 16 + lax.iota(jnp.int32, 16)
            plsc.store_scatter(idx_v, [pos], base_flat + g, mask=m)
            cvals = coords_v[pl.ds(ci * 16, 16)]
            plsc.store_scatter(cvis_v, [pos], cvals, mask=m)
            return 0

        lax.fori_loop(0, G // 16, chunk, 0)

        pltpu.sync_copy(idx_v, out_idx_hbm.at[b])
        pltpu.sync_copy(cvis_v, out_coords_hbm.at[pl.ds(b * V, V)])


def _sc_gather_body(idx_hbm, feats_hbm, out_feats_hbm,
                    idx_v, rows_v, sems):
    wid = lax.axis_index("s") * 2 + lax.axis_index("c")

    for r in range(ROWS_PER_W):
        b = wid * ROWS_PER_W + r
        pltpu.sync_copy(idx_hbm.at[b], idx_v)

        copies = [None, None]
        for ch in range(NCHUNK):
            sl = idx_v.at[pl.ds(ch * GCHUNK, GCHUNK)]
            copies[ch % 2] = pltpu.async_copy(
                feats_hbm.at[sl], rows_v.at[ch % 2], sems.at[ch % 2])
            if ch > 0:
                prev = ch - 1
                copies[prev % 2].wait()
                pltpu.sync_copy(rows_v.at[prev % 2],
                                out_feats_hbm.at[b, pl.ds(prev * GCHUNK,
                                                          GCHUNK)])
        copies[(NCHUNK - 1) % 2].wait()
        pltpu.sync_copy(rows_v.at[(NCHUNK - 1) % 2],
                        out_feats_hbm.at[b, pl.ds((NCHUNK - 1) * GCHUNK,
                                                  GCHUNK)])


def _sc_compact(posmap, coords2d):
    kern = pl.kernel(
        _sc_compact_body,
        out_type=[
            jax.ShapeDtypeStruct((B, V), jnp.int32),
            jax.ShapeDtypeStruct((B * V,), jnp.float32),
        ],
        mesh=_sc_mesh(),
        compiler_params=pltpu.CompilerParams(needs_layout_passes=False),
        scratch_types=[
            pltpu.VMEM((G,), jnp.int32),
            pltpu.VMEM((G,), jnp.float32),
            pltpu.VMEM((V,), jnp.int32),
            pltpu.VMEM((V,), jnp.float32),
        ],
    )
    return kern(posmap, coords2d)


def _sc_gather(idx_all, feats_flat):
    kern = pl.kernel(
        _sc_gather_body,
        out_type=jax.ShapeDtypeStruct((B, V, E), jnp.float32),
        mesh=_sc_mesh(),
        compiler_params=pltpu.CompilerParams(needs_layout_passes=False),
        scratch_types=[
            pltpu.VMEM((V,), jnp.int32),
            pltpu.VMEM((2, GCHUNK, E), jnp.float32),
            pltpu.SemaphoreType.DMA((2,)),
        ],
    )
    return kern(idx_all, feats_flat)


TBLK = 2048


def _mlp_body(cv_ref, w1_ref, b1_ref, w2_ref, b2_ref, g_ref, beta_ref, out_ref):
    c = cv_ref[...]
    h = c * w1_ref[...] + b1_ref[...]
    h = 0.5 * h * (1.0 + lax.erf(h * (1.0 / math.sqrt(2.0))))
    h2 = jnp.dot(h, w2_ref[...], preferred_element_type=jnp.float32)
    h2 = h2 + b2_ref[...]
    mean = jnp.mean(h2, axis=1, keepdims=True)
    ctr = h2 - mean
    var = jnp.mean(ctr * ctr, axis=1, keepdims=True)
    out_ref[...] = ctr / jnp.sqrt(var + 1e-5) * g_ref[...] + beta_ref[...]


def _mlp(cv, W1, b1, W2, b2, ln_gamma, ln_beta):
    nt = B * V
    grid = nt // TBLK
    return pl.pallas_call(
        _mlp_body,
        grid=(grid,),
        in_specs=[
            pl.BlockSpec((TBLK, 1), lambda i: (i, 0)),
            pl.BlockSpec((1, 128), lambda i: (0, 0)),
            pl.BlockSpec((1, 128), lambda i: (0, 0)),
            pl.BlockSpec((128, E), lambda i: (0, 0)),
            pl.BlockSpec((1, E), lambda i: (0, 0)),
            pl.BlockSpec((1, E), lambda i: (0, 0)),
            pl.BlockSpec((1, E), lambda i: (0, 0)),
        ],
        out_specs=pl.BlockSpec((TBLK, E), lambda i: (i, 0)),
        out_shape=jax.ShapeDtypeStruct((nt, E), jnp.float32),
    )(cv, W1, b1, W2, b2, ln_gamma, ln_beta)


def kernel(feats_emb, center_coords, rand_scores, W1, b1, W2, b2,
           ln_gamma, ln_beta):
    posmap = _posmap(rand_scores)
    coords2d = center_coords.reshape(B, G)
    feats_flat = feats_emb.reshape(B * G, E)
    idx_all, cvis = _sc_compact(posmap, coords2d)
    fv = _sc_gather(idx_all, feats_flat)
    pos_emb = _mlp(cvis.reshape(B * V, 1), W1, b1.reshape(1, 128), W2,
                   b2.reshape(1, E), ln_gamma.reshape(1, E),
                   ln_beta.reshape(1, E))
    return fv, pos_emb.reshape(B, V, E)


# speedup vs baseline: 5.6234x; 1.1386x over previous
import functools
import math

import jax
import jax.numpy as jnp
from jax import lax
from jax.experimental import pallas as pl
from jax.experimental.pallas import tpu as pltpu
from jax.experimental.pallas import tpu_sc as plsc

B, G, E = 64, 2048, 384
K = G // 2
V = G - K
NW = 32
ROWS_PER_W = B // NW
GCHUNK = 128
NCHUNK = V // GCHUNK


def _cumsum_lanes(x):
    sh = 1
    while sh < G:
        z = jnp.zeros((x.shape[0], sh), x.dtype)
        x = x + jnp.concatenate([z, x[:, :-sh]], axis=1)
        sh *= 2
    return x


def _posmap_body(scores_ref, posmap_ref):
    s = scores_ref[...]
    bits = lax.bitcast_convert_type(s, jnp.int32)

    def step(_, carry):
        lo, hi = carry
        mid = lo + ((hi - lo) >> 1)
        cnt = jnp.sum((bits > mid).astype(jnp.int32), axis=1, keepdims=True)
        pred = cnt < K
        return jnp.where(pred, lo, mid + 1), jnp.where(pred, mid, hi)

    lo0 = jnp.zeros((B, 1), jnp.int32)
    hi0 = jnp.full((B, 1), 0x7F800000, jnp.int32)
    t, _ = lax.fori_loop(0, 31, step, (lo0, hi0))

    gt = bits > t
    cnt_gt = jnp.sum(gt.astype(jnp.int32), axis=1, keepdims=True)
    need = K - cnt_gt
    tie = (bits == t).astype(jnp.int32)
    tie_excl = _cumsum_lanes(tie) - tie
    masked = gt | ((tie == 1) & (tie_excl < need))
    vis = 1 - masked.astype(jnp.int32)
    c = _cumsum_lanes(vis)
    posmap_ref[...] = jnp.where(masked, -1, c - 1)


def _posmap(rand_scores):
    return pl.pallas_call(
        _posmap_body,
        out_shape=jax.ShapeDtypeStruct((B, G), jnp.int32),
    )(rand_scores)


def _sc_mesh():
    return plsc.VectorSubcoreMesh(core_axis_name="c", subcore_axis_name="s",
                                  num_cores=2, num_subcores=16)


def _sc_compact_body(posmap_hbm, coords_hbm, out_idx_hbm, out_coords_hbm,
                     pos_v, coords_v, idx_v, cvis_v):
    wid = lax.axis_index("s") * 2 + lax.axis_index("c")

    for r in range(ROWS_PER_W):
        b = wid * ROWS_PER_W + r
        pltpu.sync_copy(posmap_hbm.at[b], pos_v)
        pltpu.sync_copy(coords_hbm.at[b], coords_v)

        base_flat = b * G

        def chunk(ci, _):
            pos = pos_v[pl.ds(ci * 16, 16)]
            m = pos >= 0
            g = ci * 16 + lax.iota(jnp.int32, 16)
            plsc.store_scatter(idx_v, [pos], base_flat + g, mask=m)
            cvals = coords_v[pl.ds(ci * 16, 16)]
            plsc.store_scatter(cvis_v, [pos], cvals, mask=m)
            return 0

        lax.fori_loop(0, G // 16, chunk, 0)

        pltpu.sync_copy(idx_v, out_idx_hbm.at[b])
        pltpu.sync_copy(cvis_v, out_coords_hbm.at[pl.ds(b * V, V)])


def _sc_gather_body(idx_hbm, feats_hbm, out_feats_hbm,
                    idx_v, rows_v, sems):
    wid = lax.axis_index("s") * 2 + lax.axis_index("c")

    for r in range(ROWS_PER_W):
        b = wid * ROWS_PER_W + r
        pltpu.sync_copy(idx_hbm.at[b], idx_v)

        copies = [None, None]
        for ch in range(NCHUNK):
            sl = idx_v.at[pl.ds(ch * GCHUNK, GCHUNK)]
            copies[ch % 2] = pltpu.async_copy(
                feats_hbm.at[sl], rows_v.at[ch % 2], sems.at[ch % 2])
            if ch > 0:
                prev = ch - 1
                copies[prev % 2].wait()
                pltpu.sync_copy(rows_v.at[prev % 2],
                                out_feats_hbm.at[b, pl.ds(prev * GCHUNK,
                                                          GCHUNK)])
        copies[(NCHUNK - 1) % 2].wait()
        pltpu.sync_copy(rows_v.at[(NCHUNK - 1) % 2],
                        out_feats_hbm.at[b, pl.ds((NCHUNK - 1) * GCHUNK,
                                                  GCHUNK)])


def _sc_compact(posmap, coords2d):
    kern = pl.kernel(
        _sc_compact_body,
        out_type=[
            jax.ShapeDtypeStruct((B, V), jnp.int32),
            jax.ShapeDtypeStruct((B * V,), jnp.float32),
        ],
        mesh=_sc_mesh(),
        compiler_params=pltpu.CompilerParams(needs_layout_passes=False),
        scratch_types=[
            pltpu.VMEM((G,), jnp.int32),
            pltpu.VMEM((G,), jnp.float32),
            pltpu.VMEM((V,), jnp.int32),
            pltpu.VMEM((V,), jnp.float32),
        ],
    )
    return kern(posmap, coords2d)


def _sc_gather(idx_all, feats_flat):
    kern = pl.kernel(
        _sc_gather_body,
        out_type=jax.ShapeDtypeStruct((B, V, E), jnp.float32),
        mesh=_sc_mesh(),
        compiler_params=pltpu.CompilerParams(needs_layout_passes=False),
        scratch_types=[
            pltpu.VMEM((V,), jnp.int32),
            pltpu.VMEM((2, GCHUNK, E), jnp.float32),
            pltpu.SemaphoreType.DMA((2,)),
        ],
    )
    return kern(idx_all, feats_flat)


RBLK = 8


def _mlp_body(cv_ref, w1c_ref, b1c_ref, w2_ref, b2_ref, g_ref, beta_ref,
              out_ref):
    w1c = w1c_ref[...]
    b1c = b1c_ref[...]
    w2 = w2_ref[...]
    b2 = b2_ref[...]
    gam = g_ref[...]
    bet = beta_ref[...]
    for s in range(RBLK):
        c_row = cv_ref[s:s + 1, :]
        ht = w1c * c_row + b1c
        ht = 0.5 * ht * (1.0 + lax.erf(ht * (1.0 / math.sqrt(2.0))))
        h2 = lax.dot_general(ht, w2, (((0,), (0,)), ((), ())),
                             preferred_element_type=jnp.float32)
        h2 = h2 + b2
        mean = jnp.mean(h2, axis=1, keepdims=True)
        ctr = h2 - mean
        var = jnp.mean(ctr * ctr, axis=1, keepdims=True)
        out_ref[pl.ds(s * V, V), :] = (
            ctr / jnp.sqrt(var + 1e-5) * gam + bet)


def _mlp(cv, W1c, b1c, W2, b2, ln_gamma, ln_beta):
    nt = B * V
    return pl.pallas_call(
        _mlp_body,
        grid=(B // RBLK,),
        in_specs=[
            pl.BlockSpec((RBLK, V), lambda i: (i, 0)),
            pl.BlockSpec((128, 1), lambda i: (0, 0)),
            pl.BlockSpec((128, 1), lambda i: (0, 0)),
            pl.BlockSpec((128, E), lambda i: (0, 0)),
            pl.BlockSpec((1, E), lambda i: (0, 0)),
            pl.BlockSpec((1, E), lambda i: (0, 0)),
            pl.BlockSpec((1, E), lambda i: (0, 0)),
        ],
        out_specs=pl.BlockSpec((RBLK * V, E), lambda i: (i, 0)),
        out_shape=jax.ShapeDtypeStruct((nt, E), jnp.float32),
    )(cv, W1c, b1c, W2, b2, ln_gamma, ln_beta)


def kernel(feats_emb, center_coords, rand_scores, W1, b1, W2, b2,
           ln_gamma, ln_beta):
    posmap = _posmap(rand_scores)
    coords2d = center_coords.reshape(B, G)
    feats_flat = feats_emb.reshape(B * G, E)
    idx_all, cvis = _sc_compact(posmap, coords2d)
    fv = _sc_gather(idx_all, feats_flat)
    pos_emb = _mlp(cvis.reshape(B, V), W1.reshape(128, 1),
                   b1.reshape(128, 1), W2, b2.reshape(1, E),
                   ln_gamma.reshape(1, E), ln_beta.reshape(1, E))
    return fv, pos_emb.reshape(B, V, E)


# speedup vs baseline: 5.6625x; 1.0070x over previous
import functools
import math

import jax
import jax.numpy as jnp
from jax import lax
from jax.experimental import pallas as pl
from jax.experimental.pallas import tpu as pltpu
from jax.experimental.pallas import tpu_sc as plsc

B, G, E = 64, 2048, 384
K = G // 2
V = G - K
NW = 32
ROWS_PER_W = B // NW
GCHUNK = 64
NCHUNK = V // GCHUNK
NBUF = 4


def _posmap_body(scores_ref, posmap_ref):
    s = scores_ref[...]
    bits = lax.bitcast_convert_type(s, jnp.int32)

    def step(_, carry):
        lo, hi = carry
        mid = lo + ((hi - lo) >> 1)
        cnt = jnp.sum((bits > mid).astype(jnp.int32), axis=1, keepdims=True)
        pred = cnt < K
        return jnp.where(pred, lo, mid + 1), jnp.where(pred, mid, hi)

    lo0 = jnp.zeros((B, 1), jnp.int32)
    hi0 = jnp.full((B, 1), 0x7F800000, jnp.int32)
    t, _ = lax.fori_loop(0, 31, step, (lo0, hi0))

    gt = (bits > t).astype(jnp.int32)
    cnt_gt = jnp.sum(gt, axis=1, keepdims=True)
    need = K - cnt_gt
    tie = (bits == t).astype(jnp.int32)
    x2 = jnp.concatenate([gt.astype(jnp.bfloat16),
                          tie.astype(jnp.bfloat16)], axis=0)
    i0 = lax.broadcasted_iota(jnp.int32, (G, G), 0)
    i1 = lax.broadcasted_iota(jnp.int32, (G, G), 1)
    m = (i0 <= i1).astype(jnp.bfloat16)
    cs = lax.dot_general(x2, m, (((1,), (0,)), ((), ())),
                         preferred_element_type=jnp.float32)
    c_gt = cs[:B].astype(jnp.int32)
    c_tie = cs[B:].astype(jnp.int32)
    masked = (gt == 1) | ((tie == 1) & (c_tie - tie < need))
    cmask = c_gt + jnp.minimum(c_tie, need)
    gidx = lax.broadcasted_iota(jnp.int32, (B, G), 1)
    posmap_ref[...] = jnp.where(masked, -1, gidx - cmask)


def _posmap(rand_scores):
    return pl.pallas_call(
        _posmap_body,
        out_shape=jax.ShapeDtypeStruct((B, G), jnp.int32),
    )(rand_scores)


def _sc_mesh():
    return plsc.VectorSubcoreMesh(core_axis_name="c", subcore_axis_name="s",
                                  num_cores=2, num_subcores=16)


def _sc_compact_body(posmap_hbm, coords_hbm, out_idx_hbm, out_coords_hbm,
                     pos_v, coords_v, idx_v, cvis_v):
    wid = lax.axis_index("s") * 2 + lax.axis_index("c")

    for r in range(ROWS_PER_W):
        b = wid * ROWS_PER_W + r
        pltpu.sync_copy(posmap_hbm.at[b], pos_v)
        pltpu.sync_copy(coords_hbm.at[b], coords_v)

        base_flat = b * G
        UNROLL = 4

        def chunk(ci, _):
            for u in range(UNROLL):
                off = ci * (16 * UNROLL) + u * 16
                pos = pos_v[pl.ds(off, 16)]
                m = pos >= 0
                g = off + lax.iota(jnp.int32, 16)
                plsc.store_scatter(idx_v, [pos], base_flat + g, mask=m)
                cvals = coords_v[pl.ds(off, 16)]
                plsc.store_scatter(cvis_v, [pos], cvals, mask=m)
            return 0

        lax.fori_loop(0, G // (16 * UNROLL), chunk, 0)

        pltpu.sync_copy(idx_v, out_idx_hbm.at[b])
        pltpu.sync_copy(cvis_v, out_coords_hbm.at[pl.ds(b * V, V)])


def _sc_gather_body(idx_hbm, feats_hbm, out_feats_hbm,
                    idx_v, rows_v, sems_g, sems_w):
    wid = lax.axis_index("s") * 2 + lax.axis_index("c")

    for r in range(ROWS_PER_W):
        b = wid * ROWS_PER_W + r
        pltpu.sync_copy(idx_hbm.at[b], idx_v)

        gathers = [None] * NBUF
        writes = [None] * NBUF
        for ch in range(NCHUNK):
            k = ch % NBUF
            if writes[k] is not None:
                writes[k].wait()
                writes[k] = None
            sl = idx_v.at[pl.ds(ch * GCHUNK, GCHUNK)]
            gathers[k] = pltpu.async_copy(
                feats_hbm.at[sl], rows_v.at[k], sems_g.at[k])
            if ch > 0:
                p = (ch - 1) % NBUF
                gathers[p].wait()
                writes[p] = pltpu.async_copy(
                    rows_v.at[p],
                    out_feats_hbm.at[b, pl.ds((ch - 1) * GCHUNK, GCHUNK)],
                    sems_w.at[p])
        last = (NCHUNK - 1) % NBUF
        gathers[last].wait()
        writes[last] = pltpu.async_copy(
            rows_v.at[last],
            out_feats_hbm.at[b, pl.ds((NCHUNK - 1) * GCHUNK, GCHUNK)],
            sems_w.at[last])
        for k in range(NBUF):
            if writes[k] is not None:
                writes[k].wait()


def _sc_compact(posmap, coords2d):
    kern = pl.kernel(
        _sc_compact_body,
        out_type=[
            jax.ShapeDtypeStruct((B, V), jnp.int32),
            jax.ShapeDtypeStruct((B * V,), jnp.float32),
        ],
        mesh=_sc_mesh(),
        compiler_params=pltpu.CompilerParams(needs_layout_passes=False),
        scratch_types=[
            pltpu.VMEM((G,), jnp.int32),
            pltpu.VMEM((G,), jnp.float32),
            pltpu.VMEM((V,), jnp.int32),
            pltpu.VMEM((V,), jnp.float32),
        ],
    )
    return kern(posmap, coords2d)


def _sc_gather(idx_all, feats_flat):
    kern = pl.kernel(
        _sc_gather_body,
        out_type=jax.ShapeDtypeStruct((B, V, E), jnp.float32),
        mesh=_sc_mesh(),
        compiler_params=pltpu.CompilerParams(needs_layout_passes=False),
        scratch_types=[
            pltpu.VMEM((V,), jnp.int32),
            pltpu.VMEM((NBUF, GCHUNK, E), jnp.float32),
            pltpu.SemaphoreType.DMA((NBUF,)),
            pltpu.SemaphoreType.DMA((NBUF,)),
        ],
    )
    return kern(idx_all, feats_flat)


RBLK = 8


def _mlp_body(cv_ref, w1c_ref, b1c_ref, w2_ref, b2_ref, g_ref, beta_ref,
              out_ref):
    w1c = w1c_ref[...]
    b1c = b1c_ref[...]
    w2 = w2_ref[...]
    b2 = b2_ref[...]
    gam = g_ref[...]
    bet = beta_ref[...]
    for s in range(RBLK):
        c_row = cv_ref[s:s + 1, :]
        ht = w1c * c_row + b1c
        ht = 0.5 * ht * (1.0 + lax.erf(ht * (1.0 / math.sqrt(2.0))))
        h2 = lax.dot_general(ht, w2, (((0,), (0,)), ((), ())),
                             preferred_element_type=jnp.float32)
        h2 = h2 + b2
        mean = jnp.mean(h2, axis=1, keepdims=True)
        ctr = h2 - mean
        var = jnp.mean(ctr * ctr, axis=1, keepdims=True)
        out_ref[pl.ds(s * V, V), :] = (
            ctr / jnp.sqrt(var + 1e-5) * gam + bet)


def _mlp(cv, W1c, b1c, W2, b2, ln_gamma, ln_beta):
    nt = B * V
    return pl.pallas_call(
        _mlp_body,
        grid=(B // RBLK,),
        in_specs=[
            pl.BlockSpec((RBLK, V), lambda i: (i, 0)),
            pl.BlockSpec((128, 1), lambda i: (0, 0)),
            pl.BlockSpec((128, 1), lambda i: (0, 0)),
            pl.BlockSpec((128, E), lambda i: (0, 0)),
            pl.BlockSpec((1, E), lambda i: (0, 0)),
            pl.BlockSpec((1, E), lambda i: (0, 0)),
            pl.BlockSpec((1, E), lambda i: (0, 0)),
        ],
        out_specs=pl.BlockSpec((RBLK * V, E), lambda i: (i, 0)),
        out_shape=jax.ShapeDtypeStruct((nt, E), jnp.float32),
    )(cv, W1c, b1c, W2, b2, ln_gamma, ln_beta)


def kernel(feats_emb, center_coords, rand_scores, W1, b1, W2, b2,
           ln_gamma, ln_beta):
    posmap = _posmap(rand_scores)
    coords2d = center_coords.reshape(B, G)
    feats_flat = feats_emb.reshape(B * G, E)
    idx_all, cvis = _sc_compact(posmap, coords2d)
    fv = _sc_gather(idx_all, feats_flat)
    pos_emb = _mlp(cvis.reshape(B, V), W1.reshape(128, 1),
                   b1.reshape(128, 1), W2, b2.reshape(1, E),
                   ln_gamma.reshape(1, E), ln_beta.reshape(1, E))
    return fv, pos_emb.reshape(B, V, E)


# speedup vs baseline: 5.6684x; 1.0010x over previous
import functools
import math

import jax
import jax.numpy as jnp
from jax import lax
from jax.experimental import pallas as pl
from jax.experimental.pallas import tpu as pltpu
from jax.experimental.pallas import tpu_sc as plsc

B, G, E = 64, 2048, 384
K = G // 2
V = G - K
NW = 32
ROWS_PER_W = B // NW
GCHUNK = 64
NCHUNK = V // GCHUNK
NBUF = 4


def _posmap_body(scores_ref, posmap_ref):
    s = scores_ref[...]
    bits = lax.bitcast_convert_type(s, jnp.int32)

    def step(_, carry):
        lo, hi = carry
        mid = lo + ((hi - lo) >> 1)
        cnt = jnp.sum((bits > mid).astype(jnp.int32), axis=1, keepdims=True)
        pred = cnt < K
        return jnp.where(pred, lo, mid + 1), jnp.where(pred, mid, hi)

    lo0 = jnp.zeros((B, 1), jnp.int32)
    hi0 = jnp.full((B, 1), 0x3F800000, jnp.int32)
    t, _ = lax.fori_loop(0, 30, step, (lo0, hi0))

    gt = (bits > t).astype(jnp.int32)
    cnt_gt = jnp.sum(gt, axis=1, keepdims=True)
    need = K - cnt_gt
    tie = (bits == t).astype(jnp.int32)
    x2 = jnp.concatenate([gt.astype(jnp.bfloat16),
                          tie.astype(jnp.bfloat16)], axis=0)
    i0 = lax.broadcasted_iota(jnp.int32, (G, G), 0)
    i1 = lax.broadcasted_iota(jnp.int32, (G, G), 1)
    m = (i0 <= i1).astype(jnp.bfloat16)
    cs = lax.dot_general(x2, m, (((1,), (0,)), ((), ())),
                         preferred_element_type=jnp.float32)
    c_gt = cs[:B].astype(jnp.int32)
    c_tie = cs[B:].astype(jnp.int32)
    masked = (gt == 1) | ((tie == 1) & (c_tie - tie < need))
    cmask = c_gt + jnp.minimum(c_tie, need)
    gidx = lax.broadcasted_iota(jnp.int32, (B, G), 1)
    posmap_ref[...] = jnp.where(masked, -1, gidx - cmask)


def _posmap(rand_scores):
    return pl.pallas_call(
        _posmap_body,
        out_shape=jax.ShapeDtypeStruct((B, G), jnp.int32),
    )(rand_scores)


def _sc_mesh():
    return plsc.VectorSubcoreMesh(core_axis_name="c", subcore_axis_name="s",
                                  num_cores=2, num_subcores=16)


def _sc_compact_body(posmap_hbm, coords_hbm, out_idx_hbm, out_coords_hbm,
                     pos_v, coords_v, idx_v, cvis_v):
    wid = lax.axis_index("s") * 2 + lax.axis_index("c")

    for r in range(ROWS_PER_W):
        b = wid * ROWS_PER_W + r
        pltpu.sync_copy(posmap_hbm.at[b], pos_v)
        pltpu.sync_copy(coords_hbm.at[b], coords_v)

        base_flat = b * G
        UNROLL = 4

        def chunk(ci, _):
            for u in range(UNROLL):
                off = ci * (16 * UNROLL) + u * 16
                pos = pos_v[pl.ds(off, 16)]
                m = pos >= 0
                g = off + lax.iota(jnp.int32, 16)
                plsc.store_scatter(idx_v, [pos], base_flat + g, mask=m)
                cvals = coords_v[pl.ds(off, 16)]
                plsc.store_scatter(cvis_v, [pos], cvals, mask=m)
            return 0

        lax.fori_loop(0, G // (16 * UNROLL), chunk, 0)

        pltpu.sync_copy(idx_v, out_idx_hbm.at[b])
        pltpu.sync_copy(cvis_v, out_coords_hbm.at[pl.ds(b * V, V)])


def _sc_gather_body(idx_hbm, feats_hbm, out_feats_hbm,
                    idx_v, rows_v, sems_g, sems_w):
    wid = lax.axis_index("s") * 2 + lax.axis_index("c")

    for r in range(ROWS_PER_W):
        b = wid * ROWS_PER_W + r
        pltpu.sync_copy(idx_hbm.at[b], idx_v)

        gathers = [None] * NBUF
        writes = [None] * NBUF
        for ch in range(NCHUNK):
            k = ch % NBUF
            if writes[k] is not None:
                writes[k].wait()
                writes[k] = None
            sl = idx_v.at[pl.ds(ch * GCHUNK, GCHUNK)]
            gathers[k] = pltpu.async_copy(
                feats_hbm.at[sl], rows_v.at[k], sems_g.at[k])
            if ch > 0:
                p = (ch - 1) % NBUF
                gathers[p].wait()
                writes[p] = pltpu.async_copy(
                    rows_v.at[p],
                    out_feats_hbm.at[b, pl.ds((ch - 1) * GCHUNK, GCHUNK)],
                    sems_w.at[p])
        last = (NCHUNK - 1) % NBUF
        gathers[last].wait()
        writes[last] = pltpu.async_copy(
            rows_v.at[last],
            out_feats_hbm.at[b, pl.ds((NCHUNK - 1) * GCHUNK, GCHUNK)],
            sems_w.at[last])
        for k in range(NBUF):
            if writes[k] is not None:
                writes[k].wait()


def _sc_compact(posmap, coords2d):
    kern = pl.kernel(
        _sc_compact_body,
        out_type=[
            jax.ShapeDtypeStruct((B, V), jnp.int32),
            jax.ShapeDtypeStruct((B * V,), jnp.float32),
        ],
        mesh=_sc_mesh(),
        compiler_params=pltpu.CompilerParams(needs_layout_passes=False),
        scratch_types=[
            pltpu.VMEM((G,), jnp.int32),
            pltpu.VMEM((G,), jnp.float32),
            pltpu.VMEM((V,), jnp.int32),
            pltpu.VMEM((V,), jnp.float32),
        ],
    )
    return kern(posmap, coords2d)


def _sc_gather(idx_all, feats_flat):
    kern = pl.kernel(
        _sc_gather_body,
        out_type=jax.ShapeDtypeStruct((B, V, E), jnp.float32),
        mesh=_sc_mesh(),
        compiler_params=pltpu.CompilerParams(needs_layout_passes=False),
        scratch_types=[
            pltpu.VMEM((V,), jnp.int32),
            pltpu.VMEM((NBUF, GCHUNK, E), jnp.float32),
            pltpu.SemaphoreType.DMA((NBUF,)),
            pltpu.SemaphoreType.DMA((NBUF,)),
        ],
    )
    return kern(idx_all, feats_flat)


RBLK = 8


def _mlp_body(cv_ref, w1_ref, b1_ref, w2_ref, b2_ref, g_ref, beta_ref,
              out_ref):
    w1c = jnp.transpose(w1_ref[...])
    b1c = jnp.transpose(b1_ref[...].reshape(1, 128))
    w2 = w2_ref[...]
    b2 = b2_ref[...].reshape(1, E)
    gam = g_ref[...].reshape(1, E)
    bet = beta_ref[...].reshape(1, E)
    for s in range(RBLK):
        c_row = cv_ref[s:s + 1, :]
        ht = w1c * c_row + b1c
        ht = 0.5 * ht * (1.0 + lax.erf(ht * (1.0 / math.sqrt(2.0))))
        h2 = lax.dot_general(ht, w2, (((0,), (0,)), ((), ())),
                             preferred_element_type=jnp.float32)
        h2 = h2 + b2
        mean = jnp.mean(h2, axis=1, keepdims=True)
        ctr = h2 - mean
        var = jnp.mean(ctr * ctr, axis=1, keepdims=True)
        out_ref[pl.ds(s * V, V), :] = (
            ctr / jnp.sqrt(var + 1e-5) * gam + bet)


def _mlp(cv, W1, b1, W2, b2, ln_gamma, ln_beta):
    nt = B * V
    return pl.pallas_call(
        _mlp_body,
        grid=(B // RBLK,),
        in_specs=[
            pl.BlockSpec((RBLK, V), lambda i: (i, 0)),
            pl.BlockSpec((1, 128), lambda i: (0, 0)),
            pl.BlockSpec((128,), lambda i: (0,)),
            pl.BlockSpec((128, E), lambda i: (0, 0)),
            pl.BlockSpec((E,), lambda i: (0,)),
            pl.BlockSpec((E,), lambda i: (0,)),
            pl.BlockSpec((E,), lambda i: (0,)),
        ],
        out_specs=pl.BlockSpec((RBLK * V, E), lambda i: (i, 0)),
        out_shape=jax.ShapeDtypeStruct((nt, E), jnp.float32),
    )(cv, W1, b1, W2, b2, ln_gamma, ln_beta)


def kernel(feats_emb, center_coords, rand_scores, W1, b1, W2, b2,
           ln_gamma, ln_beta):
    posmap = _posmap(rand_scores)
    coords2d = center_coords.reshape(B, G)
    feats_flat = feats_emb.reshape(B * G, E)
    idx_all, cvis = _sc_compact(posmap, coords2d)
    fv = _sc_gather(idx_all, feats_flat)
    pos_emb = _mlp(cvis.reshape(B, V), W1, b1, W2, b2, ln_gamma, ln_beta)
    return fv, pos_emb.reshape(B, V, E)
